# baseline (speedup 1.0000x reference)
import functools

import jax
import jax.numpy as jnp
from jax import lax
from jax.experimental import pallas as pl
from jax.experimental.pallas import tpu as pltpu
from jax.experimental.pallas import tpu_sc as plsc

_D = 256
_L = 16
_NC = 2
_NS = 16
_NW = _NC * _NS
_C = 64


def _decoder_call(total_padded: int, chunks_per_w: int):
  per_w = chunks_per_w * _C
  mesh = plsc.VectorSubcoreMesh(core_axis_name="c", subcore_axis_name="s")

  @functools.partial(
      pl.kernel,
      mesh=mesh,
      out_type=jax.ShapeDtypeStruct((total_padded,), jnp.float32),
      scratch_types=[
          pltpu.VMEM((per_w,), jnp.int32),
          pltpu.VMEM((per_w,), jnp.int32),
          pltpu.VMEM((_C, _D), jnp.float32),
          pltpu.VMEM((_C, _D), jnp.float32),
          pltpu.VMEM((_C, _D), jnp.float32),
          pltpu.VMEM((_C, _D), jnp.float32),
          pltpu.VMEM((per_w,), jnp.float32),
          pltpu.SemaphoreType.DMA,
          pltpu.SemaphoreType.DMA,
          pltpu.SemaphoreType.DMA,
          pltpu.SemaphoreType.DMA,
      ],
  )
  def decoder(xq_hbm, xa_hbm, qidx_hbm, aidx_hbm, out_hbm,
              qidx_v, aidx_v, rq0, rq1, ra0, ra1, out_v,
              sq0, sq1, sa0, sa1):
    wid = lax.axis_index("s") * _NC + lax.axis_index("c")
    base = wid * per_w
    pltpu.sync_copy(qidx_hbm.at[pl.ds(base, per_w)], qidx_v)
    pltpu.sync_copy(aidx_hbm.at[pl.ds(base, per_w)], aidx_v)

    lane = lax.iota(jnp.int32, _L)

    def fire(t, rq, ra, sq, sa):
      off = t * _C
      pltpu.async_copy(xq_hbm.at[qidx_v.at[pl.ds(off, _C)]], rq, sq)
      pltpu.async_copy(xa_hbm.at[aidx_v.at[pl.ds(off, _C)]], ra, sa)

    def drain(rq, ra, sq, sa):
      pltpu.make_async_copy(xq_hbm.at[pl.ds(0, _C)], rq, sq).wait()
      pltpu.make_async_copy(xa_hbm.at[pl.ds(0, _C)], ra, sa).wait()

    def compute(t, rq, ra):
      def group_body(g, carry):
        go = g * _L

        def one_edge(row):
          parts = []
          for p in range(4):
            pd = p * (_D // 4)
            acc = (rq[row, pl.ds(pd, _L)] * ra[row, pl.ds(pd, _L)])
            for d in range(1, _D // (4 * _L)):
              acc = acc + (rq[row, pl.ds(pd + d * _L, _L)] *
                           ra[row, pl.ds(pd + d * _L, _L)])
            parts.append(acc)
          acc = (parts[0] + parts[1]) + (parts[2] + parts[3])
          for h in (8, 4, 2, 1):
            perm = (lane + h) % _L
            acc = acc + acc.at[perm].get(mode="promise_in_bounds")
          return acc

        def edge_body(i, carry):
          rs = list(carry)
          for u in range(4):
            e = i + u * 4
            rs[u] = jnp.where(lane == e, one_edge(go + e), rs[u])
          return tuple(rs)

        zero = jnp.zeros((_L,), jnp.float32)
        r0, r1, r2, r3 = lax.fori_loop(
            0, 4, edge_body, (zero, zero, zero, zero))
        res = (r0 + r1) + (r2 + r3)
        one = jnp.float32(1.0)
        out_v[pl.ds(t * _C + go, _L)] = one / (one + jnp.exp(-res))
        return carry

      lax.fori_loop(0, _C // _L, group_body, 0)

    fire(0, rq0, ra0, sq0, sa0)

    def pair_body(i, carry):
      t0 = 2 * i
      t1 = t0 + 1
      fire(t1, rq1, ra1, sq1, sa1)
      drain(rq0, ra0, sq0, sa0)
      compute(t0, rq0, ra0)

      @pl.when(t1 + 1 < chunks_per_w)
      def _():
        fire(t1 + 1, rq0, ra0, sq0, sa0)

      drain(rq1, ra1, sq1, sa1)
      compute(t1, rq1, ra1)
      return carry

    lax.fori_loop(0, chunks_per_w // 2, pair_body, 0)
    pltpu.sync_copy(out_v, out_hbm.at[pl.ds(base, per_w)])

  return decoder


def kernel(x_question, x_answer, pos_edge_label_index, neg_edge_label_index):
  e = pos_edge_label_index.shape[1]
  total = 2 * e
  pair = 2 * _NW * _C
  total_padded = -(-total // pair) * pair
  chunks_per_w = total_padded // (_NW * _C)

  qidx = jnp.concatenate(
      [pos_edge_label_index[0], neg_edge_label_index[0]]).astype(jnp.int32)
  aidx = jnp.concatenate(
      [pos_edge_label_index[1], neg_edge_label_index[1]]).astype(jnp.int32)
  pad = total_padded - total
  qidx = jnp.pad(qidx, (0, pad))
  aidx = jnp.pad(aidx, (0, pad))

  out = _decoder_call(total_padded, chunks_per_w)(
      x_question, x_answer, qidx, aidx)
  return out[:e], out[e:total]


# speedup vs baseline: 1.3817x; 1.3817x over previous
import functools

import jax
import jax.numpy as jnp
from jax import lax
from jax.experimental import pallas as pl
from jax.experimental.pallas import tpu as pltpu
from jax.experimental.pallas import tpu_sc as plsc

_D = 256
_L = 16
_NC = 2
_NS = 16
_NW = _NC * _NS
_C = 96


def _decoder_call(total_padded: int, chunks_per_w: int):
  per_w = chunks_per_w * _C
  mesh = plsc.VectorSubcoreMesh(core_axis_name="c", subcore_axis_name="s")

  @functools.partial(
      pl.kernel,
      mesh=mesh,
      out_type=jax.ShapeDtypeStruct((total_padded,), jnp.float32),
      scratch_types=[
          pltpu.VMEM((per_w,), jnp.int32),
          pltpu.VMEM((per_w,), jnp.int32),
          pltpu.VMEM((_C, _D), jnp.float32),
          pltpu.VMEM((_C, _D), jnp.float32),
          pltpu.VMEM((_C, _D), jnp.float32),
          pltpu.VMEM((_C, _D), jnp.float32),
          pltpu.VMEM((per_w,), jnp.float32),
          pltpu.SemaphoreType.DMA,
          pltpu.SemaphoreType.DMA,
          pltpu.SemaphoreType.DMA,
          pltpu.SemaphoreType.DMA,
      ],
  )
  def decoder(xq_hbm, xa_hbm, qidx_hbm, aidx_hbm, out_hbm,
              qidx_v, aidx_v, rq0, rq1, ra0, ra1, out_v,
              sq0, sq1, sa0, sa1):
    wid = lax.axis_index("s") * _NC + lax.axis_index("c")
    base = wid * per_w
    pltpu.sync_copy(qidx_hbm.at[pl.ds(base, per_w)], qidx_v)
    pltpu.sync_copy(aidx_hbm.at[pl.ds(base, per_w)], aidx_v)

    lane = lax.iota(jnp.int32, _L)

    def fire(t, rq, ra, sq, sa):
      off = t * _C
      pltpu.async_copy(xq_hbm.at[qidx_v.at[pl.ds(off, _C)]], rq, sq)
      pltpu.async_copy(xa_hbm.at[aidx_v.at[pl.ds(off, _C)]], ra, sa)

    def drain(rq, ra, sq, sa):
      pltpu.make_async_copy(xq_hbm.at[pl.ds(0, _C)], rq, sq).wait()
      pltpu.make_async_copy(xa_hbm.at[pl.ds(0, _C)], ra, sa).wait()

    def compute(t, rq, ra):
      def group_body(g, carry):
        go = g * _L

        def one_edge(row):
          parts = []
          for p in range(4):
            pd = p * (_D // 4)
            acc = (rq[row, pl.ds(pd, _L)] * ra[row, pl.ds(pd, _L)])
            for d in range(1, _D // (4 * _L)):
              acc = acc + (rq[row, pl.ds(pd + d * _L, _L)] *
                           ra[row, pl.ds(pd + d * _L, _L)])
            parts.append(acc)
          acc = (parts[0] + parts[1]) + (parts[2] + parts[3])
          for h in (8, 4, 2, 1):
            perm = (lane + h) % _L
            acc = acc + acc.at[perm].get(mode="promise_in_bounds")
          return acc

        def edge_body(e, res):
          return jnp.where(lane == e, one_edge(go + e), res)

        res = lax.fori_loop(0, _L, edge_body, jnp.zeros((_L,), jnp.float32))
        one = jnp.float32(1.0)
        out_v[pl.ds(t * _C + go, _L)] = one / (one + jnp.exp(-res))
        return carry

      lax.fori_loop(0, _C // _L, group_body, 0)

    fire(0, rq0, ra0, sq0, sa0)

    def pair_body(i, carry):
      t0 = 2 * i
      t1 = t0 + 1
      fire(t1, rq1, ra1, sq1, sa1)
      drain(rq0, ra0, sq0, sa0)
      compute(t0, rq0, ra0)

      @pl.when(t1 + 1 < chunks_per_w)
      def _():
        fire(t1 + 1, rq0, ra0, sq0, sa0)

      drain(rq1, ra1, sq1, sa1)
      compute(t1, rq1, ra1)
      return carry

    lax.fori_loop(0, chunks_per_w // 2, pair_body, 0)
    pltpu.sync_copy(out_v, out_hbm.at[pl.ds(base, per_w)])

  return decoder


def kernel(x_question, x_answer, pos_edge_label_index, neg_edge_label_index):
  e = pos_edge_label_index.shape[1]
  total = 2 * e
  pair = 2 * _NW * _C
  total_padded = -(-total // pair) * pair
  chunks_per_w = total_padded // (_NW * _C)

  qidx = jnp.concatenate(
      [pos_edge_label_index[0], neg_edge_label_index[0]]).astype(jnp.int32)
  aidx = jnp.concatenate(
      [pos_edge_label_index[1], neg_edge_label_index[1]]).astype(jnp.int32)
  pad = total_padded - total
  qidx = jnp.pad(qidx, (0, pad))
  aidx = jnp.pad(aidx, (0, pad))

  out = _decoder_call(total_padded, chunks_per_w)(
      x_question, x_answer, qidx, aidx)
  return out[:e], out[e:total]
